# Initial kernel scaffold; baseline (speedup 1.0000x reference)
import jax
import jax.numpy as jnp
from jax.experimental import pallas as pl


def kernel(token_ids, token_table, pos_table):
    raise NotImplementedError("write your pallas kernel here")


# speedup vs baseline: 3.7222x; 3.7222x over previous
import jax
import jax.numpy as jnp
from jax import lax
from jax.experimental import pallas as pl
from jax.experimental.pallas import tpu as pltpu
from jax.experimental.pallas import tpu_sc as plsc

BATCH = 1024
SEQ = 200
D = 128
NW = 32
ROWS = BATCH * SEQ
R_PER_W = ROWS // NW
CHUNK = SEQ
N_CHUNK = R_PER_W // CHUNK
SPLIT = 128
REM = CHUNK - SPLIT


def _emb_body(ids_hbm, tok_hbm, pos_hbm, out_hbm,
              idx_a, idx_b, rows_v, pos_v, sem):
    wid = lax.axis_index("s") * 2 + lax.axis_index("c")
    base = wid * R_PER_W
    pltpu.sync_copy(pos_hbm.at[pl.ds(0, SEQ)], pos_v)

    def chunk_body(j, carry):
        off = base + j * CHUNK
        pltpu.sync_copy(ids_hbm.at[pl.ds(off, SPLIT)], idx_a)
        pltpu.sync_copy(ids_hbm.at[pl.ds(off + SPLIT, REM)], idx_b)
        c1 = pltpu.async_copy(tok_hbm.at[idx_a], rows_v.at[pl.ds(0, SPLIT)], sem)
        c2 = pltpu.async_copy(tok_hbm.at[idx_b], rows_v.at[pl.ds(SPLIT, REM)], sem)
        c1.wait()
        c2.wait()

        def add_body(r, c2_):
            for c in range(D // 16):
                sl = pl.ds(c * 16, 16)
                rows_v[r, sl] = rows_v[r, sl] + pos_v[r, sl]
            return c2_

        lax.fori_loop(0, CHUNK, add_body, 0)
        pltpu.sync_copy(rows_v, out_hbm.at[pl.ds(off, CHUNK)])
        return carry

    lax.fori_loop(0, N_CHUNK, chunk_body, 0)


@jax.jit
def _run(ids_flat, tok, pos):
    f = pl.kernel(
        _emb_body,
        mesh=plsc.VectorSubcoreMesh(core_axis_name="c", subcore_axis_name="s"),
        out_type=jax.ShapeDtypeStruct((ROWS, D), jnp.float32),
        scratch_types=[
            pltpu.VMEM((SPLIT,), jnp.int32),
            pltpu.VMEM((REM,), jnp.int32),
            pltpu.VMEM((CHUNK, D), jnp.float32),
            pltpu.VMEM((SEQ, D), jnp.float32),
            pltpu.SemaphoreType.DMA,
        ],
    )
    return f(ids_flat, tok, pos)


def kernel(token_ids, token_table, pos_table):
    ids_flat = token_ids.reshape(-1).astype(jnp.int32)
    out = _run(ids_flat, token_table, pos_table)
    return out.reshape(BATCH, SEQ, D)


# speedup vs baseline: 5.1292x; 1.3780x over previous
import jax
import jax.numpy as jnp
from jax import lax
from jax.experimental import pallas as pl
from jax.experimental.pallas import tpu as pltpu
from jax.experimental.pallas import tpu_sc as plsc

BATCH = 1024
SEQ = 200
D = 128
NW = 32
ROWS = BATCH * SEQ
R_PER_W = ROWS // NW
CHUNK = SEQ
N_CHUNK = R_PER_W // CHUNK
SPLIT = 128
REM = CHUNK - SPLIT
NBUF = 2


def _emb_body(ids_hbm, tok_hbm, pos_hbm, out_hbm,
              idx_a0, idx_b0, rows0, idx_a1, idx_b1, rows1,
              pos_v, sem0, sem1):
    idx_a = (idx_a0, idx_a1)
    idx_b = (idx_b0, idx_b1)
    rows = (rows0, rows1)
    sems = (sem0, sem1)
    wid = lax.axis_index("s") * 2 + lax.axis_index("c")
    base = wid * R_PER_W
    pltpu.sync_copy(pos_hbm.at[pl.ds(0, SEQ)], pos_v)

    def start(j, p):
        off = base + j * CHUNK
        pltpu.sync_copy(ids_hbm.at[pl.ds(off, SPLIT)], idx_a[p])
        pltpu.sync_copy(ids_hbm.at[pl.ds(off + SPLIT, REM)], idx_b[p])
        pltpu.async_copy(tok_hbm.at[idx_a[p]], rows[p].at[pl.ds(0, SPLIT)], sems[p])
        pltpu.async_copy(tok_hbm.at[idx_b[p]], rows[p].at[pl.ds(SPLIT, REM)], sems[p])

    def finish(j, p):
        pltpu.make_async_copy(tok_hbm.at[idx_a[p]], rows[p].at[pl.ds(0, SPLIT)], sems[p]).wait()
        pltpu.make_async_copy(tok_hbm.at[idx_b[p]], rows[p].at[pl.ds(SPLIT, REM)], sems[p]).wait()
        rv = rows[p]

        def add_body(r, carry):
            for c in range(D // 16):
                sl = pl.ds(c * 16, 16)
                rv[r, sl] = rv[r, sl] + pos_v[r, sl]
            return carry

        lax.fori_loop(0, CHUNK, add_body, 0)
        pltpu.sync_copy(rv, out_hbm.at[pl.ds(base + j * CHUNK, CHUNK)])

    start(0, 0)

    def body(i, carry):
        for b in range(NBUF):
            j = i * NBUF + b

            @pl.when(j + 1 < N_CHUNK)
            def _():
                start(j + 1, 1 - b)

            finish(j, b)
        return carry

    lax.fori_loop(0, N_CHUNK // NBUF, body, 0)


@jax.jit
def _run(ids_flat, tok, pos):
    f = pl.kernel(
        _emb_body,
        mesh=plsc.VectorSubcoreMesh(core_axis_name="c", subcore_axis_name="s"),
        out_type=jax.ShapeDtypeStruct((ROWS, D), jnp.float32),
        scratch_types=[
            pltpu.VMEM((SPLIT,), jnp.int32),
            pltpu.VMEM((REM,), jnp.int32),
            pltpu.VMEM((CHUNK, D), jnp.float32),
            pltpu.VMEM((SPLIT,), jnp.int32),
            pltpu.VMEM((REM,), jnp.int32),
            pltpu.VMEM((CHUNK, D), jnp.float32),
            pltpu.VMEM((SEQ, D), jnp.float32),
            pltpu.SemaphoreType.DMA,
            pltpu.SemaphoreType.DMA,
        ],
    )
    return f(ids_flat, tok, pos)


def kernel(token_ids, token_table, pos_table):
    ids_flat = token_ids.reshape(-1).astype(jnp.int32)
    out = _run(ids_flat, token_table, pos_table)
    return out.reshape(BATCH, SEQ, D)


# speedup vs baseline: 6.4181x; 1.2513x over previous
import jax
import jax.numpy as jnp
from jax import lax
from jax.experimental import pallas as pl
from jax.experimental.pallas import tpu as pltpu
from jax.experimental.pallas import tpu_sc as plsc

BATCH = 1024
SEQ = 200
D = 128
NW = 32
ROWS = BATCH * SEQ
R_PER_W = ROWS // NW
CHUNK = SEQ
N_CHUNK = R_PER_W // CHUNK
SPLIT = 128
REM = CHUNK - SPLIT
NBUF = 2


def _emb_body(ids_hbm, tok_hbm, pos_hbm, out_hbm,
              idx_v, rows0, rows1, pos_v,
              gsem0, gsem1, ssem0, ssem1):
    rows = (rows0, rows1)
    gsems = (gsem0, gsem1)
    ssems = (ssem0, ssem1)
    wid = lax.axis_index("s") * 2 + lax.axis_index("c")
    base = wid * R_PER_W
    pltpu.sync_copy(ids_hbm.at[pl.ds(base, R_PER_W)], idx_v)
    pltpu.sync_copy(pos_hbm.at[pl.ds(0, SEQ)], pos_v)

    def start(j, p):
        loc = j * CHUNK

        @pl.when(j >= NBUF)
        def _():
            pltpu.make_async_copy(rows[p], out_hbm.at[pl.ds(0, CHUNK)], ssems[p]).wait()

        pltpu.async_copy(tok_hbm.at[idx_v.at[pl.ds(loc, SPLIT)]],
                         rows[p].at[pl.ds(0, SPLIT)], gsems[p])
        pltpu.async_copy(tok_hbm.at[idx_v.at[pl.ds(loc + SPLIT, REM)]],
                         rows[p].at[pl.ds(SPLIT, REM)], gsems[p])

    def finish(j, p):
        loc = j * CHUNK
        pltpu.make_async_copy(tok_hbm.at[idx_v.at[pl.ds(loc, SPLIT)]],
                              rows[p].at[pl.ds(0, SPLIT)], gsems[p]).wait()
        pltpu.make_async_copy(tok_hbm.at[idx_v.at[pl.ds(loc + SPLIT, REM)]],
                              rows[p].at[pl.ds(SPLIT, REM)], gsems[p]).wait()
        rv = rows[p]

        def add_body(r, carry):
            for c in range(D // 16):
                sl = pl.ds(c * 16, 16)
                rv[r, sl] = rv[r, sl] + pos_v[r, sl]
            return carry

        lax.fori_loop(0, CHUNK, add_body, 0)
        pltpu.async_copy(rv, out_hbm.at[pl.ds(base + loc, CHUNK)], ssems[p])

    start(0, 0)

    def body(i, carry):
        for b in range(NBUF):
            j = i * NBUF + b

            @pl.when(j + 1 < N_CHUNK)
            def _():
                start(j + 1, 1 - b)

            finish(j, b)
        return carry

    lax.fori_loop(0, N_CHUNK // NBUF, body, 0)
    for b in range(NBUF):
        pltpu.make_async_copy(rows[b], out_hbm.at[pl.ds(0, CHUNK)], ssems[b]).wait()


@jax.jit
def _run(ids_flat, tok, pos):
    f = pl.kernel(
        _emb_body,
        mesh=plsc.VectorSubcoreMesh(core_axis_name="c", subcore_axis_name="s"),
        out_type=jax.ShapeDtypeStruct((ROWS, D), jnp.float32),
        scratch_types=[
            pltpu.VMEM((R_PER_W,), jnp.int32),
            pltpu.VMEM((CHUNK, D), jnp.float32),
            pltpu.VMEM((CHUNK, D), jnp.float32),
            pltpu.VMEM((SEQ, D), jnp.float32),
            pltpu.SemaphoreType.DMA,
            pltpu.SemaphoreType.DMA,
            pltpu.SemaphoreType.DMA,
            pltpu.SemaphoreType.DMA,
        ],
    )
    return f(ids_flat, tok, pos)


def kernel(token_ids, token_table, pos_table):
    ids_flat = token_ids.reshape(-1).astype(jnp.int32)
    out = _run(ids_flat, token_table, pos_table)
    return out.reshape(BATCH, SEQ, D)
